# Initial kernel scaffold; baseline (speedup 1.0000x reference)
import jax
import jax.numpy as jnp
from jax.experimental import pallas as pl


def kernel(xyz, features, W, b, gamma, beta):
    raise NotImplementedError("write your pallas kernel here")


# speedup vs baseline: 41.1092x; 41.1092x over previous
import functools

import jax
import jax.numpy as jnp
from jax.experimental import pallas as pl
from jax.experimental.pallas import tpu as pltpu
from jax.experimental.pallas import tpu_sc as plsc

_NPOINT = 2048
_EPS = 1e-5


def _fps_body(xyz_ref, out_ref, dist_ref):
    x = xyz_ref[0]
    y = xyz_ref[1]
    z = xyz_ref[2]
    bh, n = x.shape
    iota = jax.lax.broadcasted_iota(jnp.int32, (bh, n), 1)
    boff = (pl.program_id(0) * bh + jax.lax.broadcasted_iota(jnp.int32, (bh, 1), 0)) * n
    lane = jax.lax.broadcasted_iota(jnp.int32, (bh, 128), 1)
    dist_ref[...] = jnp.full((bh, n), 1e10, jnp.float32)

    def step(t, carry):
        far, buf = carry
        buf = jnp.where(lane == t % 128, far + boff, buf)

        @pl.when(t % 128 == 127)
        def _():
            out_ref[:, pl.ds((t // 128) * 128, 128)] = buf

        sel = iota == far
        cx = jnp.sum(jnp.where(sel, x, 0.0), axis=1, keepdims=True)
        cy = jnp.sum(jnp.where(sel, y, 0.0), axis=1, keepdims=True)
        cz = jnp.sum(jnp.where(sel, z, 0.0), axis=1, keepdims=True)
        dx = x - cx
        dy = y - cy
        dz = z - cz
        d = dx * dx + dy * dy + dz * dz
        dist = jnp.minimum(dist_ref[...], d)
        dist_ref[...] = dist
        mx = jnp.max(dist, axis=1, keepdims=True)
        cand = jnp.where(dist == mx, iota, n)
        return jnp.min(cand, axis=1, keepdims=True), buf

    jax.lax.fori_loop(
        0, _NPOINT, step,
        (jnp.zeros((bh, 1), jnp.int32), jnp.zeros((bh, 128), jnp.int32)))


def _mm_body(g_ref, w_ref, y_ref, st_ref):
    g = g_ref[0]
    w = w_ref[...]
    y = jax.lax.dot_general(w, g, (((1,), (1,)), ((), ())),
                            preferred_element_type=jnp.float32)
    y_ref[0] = y

    @pl.when(pl.program_id(0) == 0)
    def _():
        st_ref[...] = jnp.zeros_like(st_ref)

    st_ref[:, 0:1] += jnp.sum(y, axis=1, keepdims=True)
    st_ref[:, 1:2] += jnp.sum(y * y, axis=1, keepdims=True)


def _bn_body(y_ref, st_ref, gamma_ref, beta_ref, o_ref, *, batch):
    y = y_ref[0]
    count = batch * y.shape[1]
    mean = st_ref[:, 0:1] / count
    ex2 = st_ref[:, 1:2] / count
    var = jnp.maximum(ex2 - mean * mean, 0.0)
    a = gamma_ref[...] * jax.lax.rsqrt(var + _EPS)
    c = beta_ref[...] - mean * a
    o_ref[0] = jnp.maximum(y * a + c, 0.0)


def _gather_sc(f_flat, idx_flat):
    m = idx_flat.shape[1]
    c = f_flat.shape[1]
    window = 128
    mesh = plsc.VectorSubcoreMesh(core_axis_name="core", subcore_axis_name="subcore")

    @pl.kernel(out_type=jax.ShapeDtypeStruct((m, c), f_flat.dtype), mesh=mesh)
    def gather_kernel(x_hbm, i_hbm, o_hbm):
        def body(i_vmem, o_vmem):
            pltpu.sync_copy(x_hbm.at[i_vmem.at[0]], o_vmem)

        pltpu.emit_pipeline(
            body,
            grid=(m // window,),
            in_specs=[pl.BlockSpec((1, window), index_map=lambda i: (0, i))],
            out_specs=[pl.BlockSpec((window, c), index_map=lambda i: (i, 0))],
            core_axis_name=("core", "subcore"),
            dimension_semantics=(pltpu.PARALLEL,),
        )(i_hbm, o_hbm)

    return gather_kernel(f_flat, idx_flat)


def _fps_call(xyzT):
    three, b, n = xyzT.shape
    bh = b // 2
    return pl.pallas_call(
        _fps_body,
        grid=(2,),
        in_specs=[pl.BlockSpec((three, bh, n), lambda i: (0, i, 0))],
        out_specs=pl.BlockSpec((bh, _NPOINT), lambda i: (i, 0)),
        out_shape=jax.ShapeDtypeStruct((b, _NPOINT), jnp.int32),
        scratch_shapes=[pltpu.VMEM((bh, n), jnp.float32)],
        compiler_params=pltpu.CompilerParams(dimension_semantics=("parallel",)),
    )(xyzT)


def _mm_call(g, w):
    b, npoint, c = g.shape
    o = w.shape[0]
    return pl.pallas_call(
        _mm_body,
        grid=(b,),
        in_specs=[
            pl.BlockSpec((1, npoint, c), lambda i: (i, 0, 0)),
            pl.BlockSpec((o, c), lambda i: (0, 0)),
        ],
        out_specs=[
            pl.BlockSpec((1, o, npoint), lambda i: (i, 0, 0)),
            pl.BlockSpec((o, 128), lambda i: (0, 0)),
        ],
        out_shape=[
            jax.ShapeDtypeStruct((b, o, npoint), jnp.float32),
            jax.ShapeDtypeStruct((o, 128), jnp.float32),
        ],
        compiler_params=pltpu.CompilerParams(dimension_semantics=("arbitrary",)),
    )(g, w)


def _bn_call(y, st, gamma, beta):
    b, o, npoint = y.shape
    return pl.pallas_call(
        functools.partial(_bn_body, batch=b),
        grid=(b,),
        in_specs=[
            pl.BlockSpec((1, o, npoint), lambda i: (i, 0, 0)),
            pl.BlockSpec((o, 128), lambda i: (0, 0)),
            pl.BlockSpec((o, 1), lambda i: (0, 0)),
            pl.BlockSpec((o, 1), lambda i: (0, 0)),
        ],
        out_specs=pl.BlockSpec((1, o, npoint), lambda i: (i, 0, 0)),
        out_shape=jax.ShapeDtypeStruct((b, o, npoint), jnp.float32),
        compiler_params=pltpu.CompilerParams(dimension_semantics=("parallel",)),
    )(y, st, gamma, beta)


def kernel(xyz, features, W, b, gamma, beta):
    del b
    B, N, _ = xyz.shape
    C = features.shape[1]
    O = W.shape[0]

    xyzT = jnp.transpose(xyz, (2, 0, 1))
    idx = _fps_call(xyzT)
    f_flat = jnp.transpose(features, (0, 2, 1)).reshape(B * N, C)
    g = _gather_sc(f_flat, idx.reshape(1, B * _NPOINT)).reshape(B, _NPOINT, C)
    y, st = _mm_call(g, W)
    return _bn_call(y, st, gamma.reshape(O, 1), beta.reshape(O, 1))


# speedup vs baseline: 63.7691x; 1.5512x over previous
import functools

import jax
import jax.numpy as jnp
from jax.experimental import pallas as pl
from jax.experimental.pallas import tpu as pltpu
from jax.experimental.pallas import tpu_sc as plsc

_NPOINT = 2048
_EPS = 1e-5


_FPS_CHUNK = 512


def _fps_body(xyz_ref, out_ref, dist_ref):
    three, bh, n = xyz_ref.shape
    ch = _FPS_CHUNK
    nch = n // ch
    row = jax.lax.broadcasted_iota(jnp.int32, (bh, 1), 0)
    boff = (pl.program_id(0) * bh + row) * n
    lane128 = jax.lax.broadcasted_iota(jnp.int32, (bh, 128), 1)
    iota_c = jax.lax.broadcasted_iota(jnp.int32, (bh, ch), 1)
    dist_ref[...] = jnp.full((bh, n), 1e10, jnp.float32)

    def step(t, carry):
        far, cx, cy, cz, buf = carry
        buf = jnp.where(lane128 == t % 128, far + boff, buf)

        @pl.when(t % 128 == 127)
        def _():
            out_ref[:, pl.ds((t // 128) * 128, 128)] = buf

        macc = jnp.full((bh, ch), -jnp.inf, jnp.float32)
        iacc = jnp.zeros((bh, ch), jnp.int32)
        xacc = jnp.zeros((bh, ch), jnp.float32)
        yacc = jnp.zeros((bh, ch), jnp.float32)
        zacc = jnp.zeros((bh, ch), jnp.float32)
        for c in range(nch):
            sl = slice(c * ch, (c + 1) * ch)
            xc = xyz_ref[0, :, sl]
            yc = xyz_ref[1, :, sl]
            zc = xyz_ref[2, :, sl]
            dx = xc - cx
            dy = yc - cy
            dz = zc - cz
            d = dx * dx + dy * dy + dz * dz
            dmin = jnp.minimum(dist_ref[:, sl], d)
            dist_ref[:, sl] = dmin
            gt = dmin > macc
            macc = jnp.where(gt, dmin, macc)
            iacc = jnp.where(gt, iota_c + (c * ch), iacc)
            xacc = jnp.where(gt, xc, xacc)
            yacc = jnp.where(gt, yc, yacc)
            zacc = jnp.where(gt, zc, zacc)
        mx = jnp.max(macc, axis=1, keepdims=True)
        cand = jnp.where(macc == mx, iacc, n)
        amax = jnp.min(cand, axis=1, keepdims=True)
        sel = cand == amax
        ncx = jnp.sum(jnp.where(sel, xacc, 0.0), axis=1, keepdims=True)
        ncy = jnp.sum(jnp.where(sel, yacc, 0.0), axis=1, keepdims=True)
        ncz = jnp.sum(jnp.where(sel, zacc, 0.0), axis=1, keepdims=True)
        return amax, ncx, ncy, ncz, buf

    jax.lax.fori_loop(
        0, _NPOINT, step,
        (jnp.zeros((bh, 1), jnp.int32),
         xyz_ref[0, :, 0:1], xyz_ref[1, :, 0:1], xyz_ref[2, :, 0:1],
         jnp.zeros((bh, 128), jnp.int32)))


def _mm_body(g_ref, w_ref, y_ref, st_ref):
    g = g_ref[0]
    w = w_ref[...]
    y = jax.lax.dot_general(w, g, (((1,), (1,)), ((), ())),
                            preferred_element_type=jnp.float32)
    y_ref[0] = y

    @pl.when(pl.program_id(0) == 0)
    def _():
        st_ref[...] = jnp.zeros_like(st_ref)

    st_ref[:, 0:1] += jnp.sum(y, axis=1, keepdims=True)
    st_ref[:, 1:2] += jnp.sum(y * y, axis=1, keepdims=True)


def _bn_body(y_ref, st_ref, gamma_ref, beta_ref, o_ref, *, batch):
    y = y_ref[0]
    count = batch * y.shape[1]
    mean = st_ref[:, 0:1] / count
    ex2 = st_ref[:, 1:2] / count
    var = jnp.maximum(ex2 - mean * mean, 0.0)
    a = gamma_ref[...] * jax.lax.rsqrt(var + _EPS)
    c = beta_ref[...] - mean * a
    o_ref[0] = jnp.maximum(y * a + c, 0.0)


def _gather_sc(f_flat, idx_flat):
    m = idx_flat.shape[1]
    c = f_flat.shape[1]
    window = 128
    mesh = plsc.VectorSubcoreMesh(core_axis_name="core", subcore_axis_name="subcore")

    @pl.kernel(out_type=jax.ShapeDtypeStruct((m, c), f_flat.dtype), mesh=mesh)
    def gather_kernel(x_hbm, i_hbm, o_hbm):
        def body(i_vmem, o_vmem):
            pltpu.sync_copy(x_hbm.at[i_vmem.at[0]], o_vmem)

        pltpu.emit_pipeline(
            body,
            grid=(m // window,),
            in_specs=[pl.BlockSpec((1, window), index_map=lambda i: (0, i))],
            out_specs=[pl.BlockSpec((window, c), index_map=lambda i: (i, 0))],
            core_axis_name=("core", "subcore"),
            dimension_semantics=(pltpu.PARALLEL,),
        )(i_hbm, o_hbm)

    return gather_kernel(f_flat, idx_flat)


def _fps_call(xyzT):
    three, b, n = xyzT.shape
    bh = b // 2
    return pl.pallas_call(
        _fps_body,
        grid=(2,),
        in_specs=[pl.BlockSpec((three, bh, n), lambda i: (0, i, 0))],
        out_specs=pl.BlockSpec((bh, _NPOINT), lambda i: (i, 0)),
        out_shape=jax.ShapeDtypeStruct((b, _NPOINT), jnp.int32),
        scratch_shapes=[pltpu.VMEM((bh, n), jnp.float32)],
        compiler_params=pltpu.CompilerParams(dimension_semantics=("parallel",)),
    )(xyzT)


def _mm_call(g, w):
    b, npoint, c = g.shape
    o = w.shape[0]
    return pl.pallas_call(
        _mm_body,
        grid=(b,),
        in_specs=[
            pl.BlockSpec((1, npoint, c), lambda i: (i, 0, 0)),
            pl.BlockSpec((o, c), lambda i: (0, 0)),
        ],
        out_specs=[
            pl.BlockSpec((1, o, npoint), lambda i: (i, 0, 0)),
            pl.BlockSpec((o, 128), lambda i: (0, 0)),
        ],
        out_shape=[
            jax.ShapeDtypeStruct((b, o, npoint), jnp.float32),
            jax.ShapeDtypeStruct((o, 128), jnp.float32),
        ],
        compiler_params=pltpu.CompilerParams(dimension_semantics=("arbitrary",)),
    )(g, w)


def _bn_call(y, st, gamma, beta):
    b, o, npoint = y.shape
    return pl.pallas_call(
        functools.partial(_bn_body, batch=b),
        grid=(b,),
        in_specs=[
            pl.BlockSpec((1, o, npoint), lambda i: (i, 0, 0)),
            pl.BlockSpec((o, 128), lambda i: (0, 0)),
            pl.BlockSpec((o, 1), lambda i: (0, 0)),
            pl.BlockSpec((o, 1), lambda i: (0, 0)),
        ],
        out_specs=pl.BlockSpec((1, o, npoint), lambda i: (i, 0, 0)),
        out_shape=jax.ShapeDtypeStruct((b, o, npoint), jnp.float32),
        compiler_params=pltpu.CompilerParams(dimension_semantics=("parallel",)),
    )(y, st, gamma, beta)


def kernel(xyz, features, W, b, gamma, beta):
    del b
    B, N, _ = xyz.shape
    C = features.shape[1]
    O = W.shape[0]

    xyzT = jnp.transpose(xyz, (2, 0, 1))
    idx = _fps_call(xyzT)
    f_flat = jnp.transpose(features, (0, 2, 1)).reshape(B * N, C)
    g = _gather_sc(f_flat, idx.reshape(1, B * _NPOINT)).reshape(B, _NPOINT, C)
    y, st = _mm_call(g, W)
    return _bn_call(y, st, gamma.reshape(O, 1), beta.reshape(O, 1))
